# baseline (speedup 1.0000x reference)
import numpy as np
import jax
import jax.numpy as jnp
from jax import lax
from jax.experimental import pallas as pl
from jax.experimental.pallas import tpu as pltpu
from jax.experimental.pallas import tpu_sc as plsc

N_TOK = 32768
D = 768
NO = 10
NC = 2
NS = 16
NW = NC * NS
TPW = N_TOK // NW
TILE = 64
NTILES = TPW // TILE
NCH = D // 16
FLAT = TILE * NO


def _round_bf16(v):
    b = lax.bitcast_convert_type(v, jnp.int32)
    lsb = jnp.bitwise_and(lax.shift_right_logical(b, 16), 1)
    r = jnp.bitwise_and(b + 0x7FFF + lsb, jnp.int32(-65536))
    return lax.bitcast_convert_type(r, jnp.float32)


def _sc_body(x_hbm, wt_hbm, b_hbm, out_hbm, y_hbm, conf_hbm,
             xbuf, wbuf, bbuf, rowstage, ystage, ycomp, confbuf):
    wid = lax.axis_index("s") * NC + lax.axis_index("c")
    base_w = wid * TPW

    pltpu.sync_copy(wt_hbm, wbuf)
    pltpu.sync_copy(b_hbm, bbuf)

    ii = lax.iota(jnp.int32, 16)

    def tile_body(t_i, _):
        base = base_w + t_i * TILE
        pltpu.sync_copy(x_hbm.at[pl.ds(base, TILE)], xbuf)
        bvec = bbuf[...]

        def tok_body(t, _):
            accs = [jnp.zeros((16,), jnp.float32) for _ in range(NO + 1)]
            for cch in range(NCH):
                sl = pl.ds(cch * 16, 16)
                xb = _round_bf16(xbuf[t, sl])
                for j in range(NO + 1):
                    accs[j] = accs[j] + xb * wbuf[j, sl]
            z = bvec
            for j in range(NO + 1):
                z = z + jnp.where(ii == j, jnp.sum(accs[j]), 0.0)
            confrow = 1.0 / (1.0 + jnp.exp(-z))
            g_t = z[NO]
            exited = g_t > 0.0
            yf = jnp.where(exited, 1.0, 0.0)
            xf = 1.0 - yf
            ystage[t, :] = z * yf
            rowstage[t, :] = confrow
            for cch in range(NCH):
                sl = pl.ds(cch * 16, 16)
                xbuf[t, sl] = xbuf[t, sl] * xf
            return 0

        lax.fori_loop(0, TILE, tok_body, 0)

        for k in range(FLAT // 16):
            p0 = 16 * k
            acc = jnp.zeros((16,), jnp.float32)
            r_lo = p0 // NO
            r_hi = (p0 + 15) // NO
            pos = ii + p0
            idx = lax.rem(pos, NO)
            rowid = lax.div(pos, NO)
            dnums = lax.GatherDimensionNumbers(
                offset_dims=(), collapsed_slice_dims=(0,),
                start_index_map=(0,))
            for r in range(r_lo, r_hi + 1):
                m = rowid == r
                row = ystage[r, :]
                g = lax.gather(row, idx[:, None], dimension_numbers=dnums,
                               slice_sizes=(1,),
                               mode=lax.GatherScatterMode.PROMISE_IN_BOUNDS)
                acc = acc + jnp.where(m, g, 0.0)
            ycomp[pl.ds(p0, 16)] = acc

        for k in range(TILE // 16):
            cv = jnp.zeros((16,), jnp.float32)
            for tt in range(16):
                row = rowstage[k * 16 + tt, :]
                cv = cv + jnp.where(ii == tt, row[NO], 0.0)
            confbuf[pl.ds(k * 16, 16)] = cv

        pltpu.sync_copy(xbuf, out_hbm.at[pl.ds(base, TILE)])
        pltpu.sync_copy(ycomp, y_hbm.at[pl.ds(base * NO, FLAT)])
        pltpu.sync_copy(confbuf, conf_hbm.at[pl.ds(base, TILE)])
        return 0

    lax.fori_loop(0, NTILES, tile_body, 0)


def kernel(X, Wg, bg, Wc, bc):
    w_all_t = jnp.concatenate([Wc.T, Wg.T], axis=0)
    w_all_t = w_all_t.astype(jnp.bfloat16).astype(jnp.float32)
    b_vec = jnp.zeros((16,), jnp.float32)
    b_vec = b_vec.at[:NO].set(bc).at[NO].set(bg[0])

    mesh = plsc.VectorSubcoreMesh(core_axis_name="c", subcore_axis_name="s")
    run = pl.kernel(
        _sc_body,
        mesh=mesh,
        compiler_params=pltpu.CompilerParams(needs_layout_passes=False),
        out_type=[
            jax.ShapeDtypeStruct((N_TOK, D), jnp.float32),
            jax.ShapeDtypeStruct((N_TOK * NO,), jnp.float32),
            jax.ShapeDtypeStruct((N_TOK,), jnp.float32),
        ],
        scratch_types=[
            pltpu.VMEM((TILE, D), jnp.float32),
            pltpu.VMEM((NO + 1, D), jnp.float32),
            pltpu.VMEM((16,), jnp.float32),
            pltpu.VMEM((TILE, 16), jnp.float32),
            pltpu.VMEM((TILE, 16), jnp.float32),
            pltpu.VMEM((FLAT,), jnp.float32),
            pltpu.VMEM((TILE,), jnp.float32),
        ],
    )
    out, y_flat, conf = run(X, w_all_t, b_vec)
    return out, y_flat.reshape(N_TOK, NO), conf


# speedup vs baseline: 8.6707x; 8.6707x over previous
import jax
import jax.numpy as jnp
from jax.experimental import pallas as pl

N_TOK = 32768
D = 768
NUM_OUTPUTS = 10
BLK = 4096


def _fused_body(x_ref, w_ref, b_ref, out_ref, y_ref, conf_ref):
    x = x_ref[...]
    z = jnp.dot(x, w_ref[...], preferred_element_type=jnp.float32)
    z = z + b_ref[...]
    v = z[:, 0:1]
    conf = jax.nn.sigmoid(v)
    mask = conf > 0.5
    conf_ref[...] = conf
    y_ref[...] = jnp.where(mask, z[:, 1:], 0.0)
    out_ref[...] = jnp.where(mask, 0.0, x)


def kernel(X, Wg, bg, Wc, bc):
    w_all = jnp.concatenate([Wg, Wc], axis=1)
    b_all = jnp.concatenate([bg, bc]).reshape(1, 1 + NUM_OUTPUTS)

    grid = (N_TOK // BLK,)
    out, y_hat, conf = pl.pallas_call(
        _fused_body,
        grid=grid,
        in_specs=[
            pl.BlockSpec((BLK, D), lambda i: (i, 0)),
            pl.BlockSpec((D, 1 + NUM_OUTPUTS), lambda i: (0, 0)),
            pl.BlockSpec((1, 1 + NUM_OUTPUTS), lambda i: (0, 0)),
        ],
        out_specs=[
            pl.BlockSpec((BLK, D), lambda i: (i, 0)),
            pl.BlockSpec((BLK, NUM_OUTPUTS), lambda i: (i, 0)),
            pl.BlockSpec((BLK, 1), lambda i: (i, 0)),
        ],
        out_shape=[
            jax.ShapeDtypeStruct((N_TOK, D), jnp.float32),
            jax.ShapeDtypeStruct((N_TOK, NUM_OUTPUTS), jnp.float32),
            jax.ShapeDtypeStruct((N_TOK, 1), jnp.float32),
        ],
    )(X, w_all, b_all)
    return out, y_hat, conf.reshape(-1)
